# Initial kernel scaffold; baseline (speedup 1.0000x reference)
import jax
import jax.numpy as jnp
from jax.experimental import pallas as pl


def kernel(in_values, active_in_indices, active_label_indices, W1, b1, W2, b2):
    raise NotImplementedError("write your pallas kernel here")


# speedup vs baseline: 16.3469x; 16.3469x over previous
import functools

import jax
import jax.numpy as jnp
from jax import lax
from jax.experimental import pallas as pl
from jax.experimental.pallas import tpu as pltpu
from jax.experimental.pallas import tpu_sc as plsc

NC, NS, LANES = 2, 16, 16


@functools.lru_cache(maxsize=None)
def _build_sc_net(B, A_IN, L_OUT, H, F, C):
    NW = NC * NS
    SPT = B // NW
    NCH = H // LANES
    KCH = L_OUT // 128

    mesh = plsc.VectorSubcoreMesh(core_axis_name="c", subcore_axis_name="s")

    @functools.partial(
        pl.kernel,
        out_type=jax.ShapeDtypeStruct((B, L_OUT), jnp.float32),
        mesh=mesh,
        compiler_params=pltpu.CompilerParams(needs_layout_passes=False),
        scratch_types=[
            pltpu.VMEM((SPT, A_IN), jnp.int32),
            pltpu.VMEM((SPT * KCH, 128), jnp.int32),
            pltpu.VMEM((SPT, A_IN), jnp.float32),
            pltpu.VMEM((H,), jnp.float32),
            pltpu.VMEM((SPT, H), jnp.float32),
            pltpu.VMEM((128, H), jnp.float32),
            pltpu.VMEM((128,), jnp.float32),
            pltpu.VMEM((LANES * LANES,), jnp.float32),
            pltpu.VMEM((L_OUT,), jnp.float32),
            pltpu.SemaphoreType.DMA,
        ],
    )
    def sc_net(inv_hbm, idx1_hbm, idx2_hbm, w1t_hbm, b1_hbm, w2_hbm, b2_hbm,
               out_hbm, idx1_v, idx2_v, inv_v, b1_v, val1_v, rows_v, bsel_v,
               red_v, val2_v, sem):
        cid = lax.axis_index("c")
        sid = lax.axis_index("s")
        wid = sid * NC + cid
        base = wid * SPT

        pltpu.sync_copy(idx1_hbm.at[pl.ds(base, SPT)], idx1_v)
        pltpu.sync_copy(idx2_hbm.at[pl.ds(base * KCH, SPT * KCH)], idx2_v)
        pltpu.sync_copy(inv_hbm.at[pl.ds(base, SPT)], inv_v)
        pltpu.sync_copy(b1_hbm, b1_v)

        iota = lax.iota(jnp.int32, LANES)

        def sample_body(b, carry):
            pltpu.async_copy(w1t_hbm.at[idx1_v.at[b]], rows_v, sem).wait()

            def a_body(a0, accs):
                wvec = inv_v[b, pl.ds(a0 * LANES, LANES)]
                for j in range(LANES):
                    w = wvec[j]
                    accs = tuple(
                        accs[ci] + w * rows_v[a0 * LANES + j,
                                              pl.ds(ci * LANES, LANES)]
                        for ci in range(NCH)
                    )
                return accs

            accs = lax.fori_loop(
                0, A_IN // LANES, a_body,
                tuple(jnp.zeros((LANES,), jnp.float32) for _ in range(NCH)),
            )
            for ci in range(NCH):
                val1_v[b, pl.ds(ci * LANES, LANES)] = jnp.maximum(
                    accs[ci] + b1_v[pl.ds(ci * LANES, LANES)], 0.0)

            v1 = [val1_v[b, pl.ds(ci * LANES, LANES)] for ci in range(NCH)]

            def chunk_body(k, carry2):
                idx_row = idx2_v.at[b * KCH + k]
                pltpu.async_copy(w2_hbm.at[idx_row], rows_v, sem).wait()
                pltpu.async_copy(b2_hbm.at[idx_row], bsel_v, sem).wait()

                def grp_body(g, carry3):
                    for p in range(LANES):
                        row = g * LANES + p
                        acc = v1[0] * rows_v[row, pl.ds(0, LANES)]
                        for ci in range(1, NCH):
                            acc = acc + v1[ci] * rows_v[
                                row, pl.ds(ci * LANES, LANES)]
                        red_v[pl.ds(p * LANES, LANES)] = acc
                    iota16 = iota * LANES
                    vs = plsc.load_gather(red_v, [iota16])
                    for h in range(1, LANES):
                        vs = vs + plsc.load_gather(red_v, [iota16 + h])
                    vs = vs + bsel_v[pl.ds(g * LANES, LANES)]
                    val2_v[pl.ds(k * 128 + g * LANES, LANES)] = vs
                    return carry3

                lax.fori_loop(0, 128 // LANES, grp_body, 0)
                return carry2

            lax.fori_loop(0, KCH, chunk_body, 0)
            pltpu.sync_copy(val2_v, out_hbm.at[base + b])
            return carry

        lax.fori_loop(0, SPT, sample_body, 0)

    return sc_net


def kernel(in_values, active_in_indices, active_label_indices, W1, b1, W2, b2):
    B, A_IN = in_values.shape
    L_OUT = active_label_indices.shape[1]
    H, F = W1.shape
    C = W2.shape[0]

    idx1 = active_in_indices.astype(jnp.int32)
    idx2 = active_label_indices.astype(jnp.int32).reshape(B * (L_OUT // 128), 128)
    w1t = W1.T

    sc_net = _build_sc_net(B, A_IN, L_OUT, H, F, C)
    val2 = sc_net(in_values, idx1, idx2, w1t, b1, W2, b2)
    return (val2, active_label_indices)


# speedup vs baseline: 26.7011x; 1.6334x over previous
import functools

import jax
import jax.numpy as jnp
from jax import lax
from jax.experimental import pallas as pl
from jax.experimental.pallas import tpu as pltpu
from jax.experimental.pallas import tpu_sc as plsc

NC, NS, LANES = 2, 16, 16


@functools.lru_cache(maxsize=None)
def _build_sc_net(B, A_IN, L_OUT, H, F, C):
    NW = NC * NS
    SPT = B // NW
    NCH = H // LANES
    KCH = L_OUT // 128

    mesh = plsc.VectorSubcoreMesh(core_axis_name="c", subcore_axis_name="s")

    @functools.partial(
        pl.kernel,
        out_type=jax.ShapeDtypeStruct((B, L_OUT), jnp.float32),
        mesh=mesh,
        compiler_params=pltpu.CompilerParams(needs_layout_passes=False),
        scratch_types=[
            pltpu.VMEM((SPT, A_IN), jnp.int32),
            pltpu.VMEM((2, KCH, 128), jnp.int32),
            pltpu.VMEM((SPT, A_IN), jnp.float32),
            pltpu.VMEM((H,), jnp.float32),
            pltpu.VMEM((SPT, H), jnp.float32),
            pltpu.VMEM((A_IN, H), jnp.float32),
            pltpu.VMEM((2, 128, H), jnp.float32),
            pltpu.VMEM((KCH, 128), jnp.float32),
            pltpu.VMEM((LANES * LANES,), jnp.float32),
            pltpu.VMEM((L_OUT,), jnp.float32),
            pltpu.SemaphoreType.DMA,
            pltpu.SemaphoreType.DMA,
            pltpu.SemaphoreType.DMA,
            pltpu.SemaphoreType.DMA,
            pltpu.SemaphoreType.DMA,
        ],
    )
    def sc_net(inv_hbm, idx1_hbm, idx2_hbm, w1t_hbm, b1_hbm, w2_hbm, b2_hbm,
               out_hbm, idx1_v, idx2_v, inv_v, b1_v, val1_v, rows1_v, rows2_v,
               bsel_v, red_v, val2_v, sem1, semr0, semr1, semb, semi):
        cid = lax.axis_index("c")
        sid = lax.axis_index("s")
        wid = sid * NC + cid
        base = wid * SPT

        pltpu.sync_copy(idx1_hbm.at[pl.ds(base, SPT)], idx1_v)
        pltpu.sync_copy(inv_hbm.at[pl.ds(base, SPT)], inv_v)
        pltpu.sync_copy(b1_hbm, b1_v)

        iota = lax.iota(jnp.int32, LANES)
        semr = [semr0, semr1]

        pltpu.async_copy(idx2_hbm.at[pl.ds(base * KCH, KCH)], idx2_v.at[0],
                         semi)
        pltpu.async_copy(w1t_hbm.at[idx1_v.at[0]], rows1_v, sem1)

        def sample_body(b, carry):
            par = lax.rem(b, 2)
            nxt = lax.rem(b + 1, 2)

            pltpu.make_async_copy(
                idx2_hbm.at[pl.ds((base + b) * KCH, KCH)], idx2_v.at[par],
                semi).wait()

            @pl.when(b + 1 < SPT)
            def _():
                pltpu.async_copy(
                    idx2_hbm.at[pl.ds((base + b + 1) * KCH, KCH)],
                    idx2_v.at[nxt], semi)

            bcps = [
                pltpu.async_copy(b2_hbm.at[idx2_v.at[par, k]], bsel_v.at[k],
                                 semb)
                for k in range(KCH)
            ]
            pltpu.async_copy(w2_hbm.at[idx2_v.at[par, 0]], rows2_v.at[0],
                             semr[0])

            pltpu.make_async_copy(w1t_hbm.at[idx1_v.at[b]], rows1_v,
                                  sem1).wait()

            def a_body(a0, accs):
                wvec = inv_v[b, pl.ds(a0 * LANES, LANES)]
                for j in range(LANES):
                    w = wvec[j]
                    accs = tuple(
                        accs[ci] + w * rows1_v[a0 * LANES + j,
                                               pl.ds(ci * LANES, LANES)]
                        for ci in range(NCH)
                    )
                return accs

            accs = lax.fori_loop(
                0, A_IN // LANES, a_body,
                tuple(jnp.zeros((LANES,), jnp.float32) for _ in range(NCH)),
            )
            v1 = []
            for ci in range(NCH):
                v = jnp.maximum(accs[ci] + b1_v[pl.ds(ci * LANES, LANES)], 0.0)
                val1_v[b, pl.ds(ci * LANES, LANES)] = v
                v1.append(v)

            for cp in bcps:
                cp.wait()

            for k in range(KCH):
                if k + 1 < KCH:
                    pltpu.async_copy(w2_hbm.at[idx2_v.at[par, k + 1]],
                                     rows2_v.at[(k + 1) % 2], semr[(k + 1) % 2])
                else:
                    @pl.when(b + 1 < SPT)
                    def _():
                        pltpu.async_copy(w1t_hbm.at[idx1_v.at[b + 1]], rows1_v,
                                         sem1)
                pltpu.make_async_copy(w2_hbm.at[idx2_v.at[par, k]],
                                      rows2_v.at[k % 2], semr[k % 2]).wait()

                def grp_body(g, carry3, _k=k):
                    for p in range(LANES):
                        row = g * LANES + p
                        acc = v1[0] * rows2_v[_k % 2, row, pl.ds(0, LANES)]
                        for ci in range(1, NCH):
                            acc = acc + v1[ci] * rows2_v[
                                _k % 2, row, pl.ds(ci * LANES, LANES)]
                        red_v[pl.ds(p * LANES, LANES)] = acc
                    iota16 = iota * LANES
                    vs = plsc.load_gather(red_v, [iota16])
                    for h in range(1, LANES):
                        vs = vs + plsc.load_gather(red_v, [iota16 + h])
                    vs = vs + bsel_v[_k, pl.ds(g * LANES, LANES)]
                    val2_v[pl.ds(_k * 128 + g * LANES, LANES)] = vs
                    return carry3

                lax.fori_loop(0, 128 // LANES, grp_body, 0)

            pltpu.sync_copy(val2_v, out_hbm.at[base + b])
            return carry

        lax.fori_loop(0, SPT, sample_body, 0)

    return sc_net


def kernel(in_values, active_in_indices, active_label_indices, W1, b1, W2, b2):
    B, A_IN = in_values.shape
    L_OUT = active_label_indices.shape[1]
    H, F = W1.shape
    C = W2.shape[0]

    idx1 = active_in_indices.astype(jnp.int32)
    idx2 = active_label_indices.astype(jnp.int32).reshape(B * (L_OUT // 128), 128)
    w1t = W1.T

    sc_net = _build_sc_net(B, A_IN, L_OUT, H, F, C)
    val2 = sc_net(in_values, idx1, idx2, w1t, b1, W2, b2)
    return (val2, active_label_indices)


# speedup vs baseline: 35.3896x; 1.3254x over previous
import functools

import jax
import jax.numpy as jnp
from jax import lax
from jax.experimental import pallas as pl
from jax.experimental.pallas import tpu as pltpu
from jax.experimental.pallas import tpu_sc as plsc

NC, NS, LANES = 2, 16, 16


def _tree_sum(vals):
    vals = list(vals)
    while len(vals) > 1:
        nxt = [vals[i] + vals[i + 1] for i in range(0, len(vals) - 1, 2)]
        if len(vals) % 2:
            nxt.append(vals[-1])
        vals = nxt
    return vals[0]


@functools.lru_cache(maxsize=None)
def _build_sc_net(B, A_IN, L_OUT, H, F, C):
    NW = NC * NS
    SPT = B // NW
    NCH = H // LANES
    KCH = L_OUT // 128
    NG = 128 // LANES

    mesh = plsc.VectorSubcoreMesh(core_axis_name="c", subcore_axis_name="s")

    @functools.partial(
        pl.kernel,
        out_type=jax.ShapeDtypeStruct((B, L_OUT), jnp.float32),
        mesh=mesh,
        compiler_params=pltpu.CompilerParams(needs_layout_passes=False),
        scratch_types=[
            pltpu.VMEM((SPT, A_IN), jnp.int32),
            pltpu.VMEM((2, KCH, 128), jnp.int32),
            pltpu.VMEM((SPT, A_IN), jnp.float32),
            pltpu.VMEM((H,), jnp.float32),
            pltpu.VMEM((A_IN, H), jnp.float32),
            pltpu.VMEM((2, 128, H), jnp.float32),
            pltpu.VMEM((KCH, 128), jnp.float32),
            pltpu.VMEM((NG * LANES * LANES,), jnp.float32),
            pltpu.VMEM((2, L_OUT), jnp.float32),
            pltpu.SemaphoreType.DMA,
            pltpu.SemaphoreType.DMA,
            pltpu.SemaphoreType.DMA,
            pltpu.SemaphoreType.DMA,
            pltpu.SemaphoreType.DMA,
            pltpu.SemaphoreType.DMA,
        ],
    )
    def sc_net(inv_hbm, idx1_hbm, idx2_hbm, w1t_hbm, b1_hbm, w2_hbm, b2_hbm,
               out_hbm, idx1_v, idx2_v, inv_v, b1_v, rows1_v, rows2_v,
               bsel_v, red_v, val2_v, sem1, semr0, semr1, semb, semi, semo):
        cid = lax.axis_index("c")
        sid = lax.axis_index("s")
        wid = sid * NC + cid
        base = wid * SPT

        pltpu.sync_copy(idx1_hbm.at[pl.ds(base, SPT)], idx1_v)
        pltpu.sync_copy(inv_hbm.at[pl.ds(base, SPT)], inv_v)
        pltpu.sync_copy(b1_hbm, b1_v)

        iota = lax.iota(jnp.int32, LANES)
        semr = [semr0, semr1]

        pltpu.async_copy(idx2_hbm.at[pl.ds(base * KCH, KCH)], idx2_v.at[0],
                         semi)
        pltpu.async_copy(w1t_hbm.at[idx1_v.at[0]], rows1_v, sem1)

        def sample_body(b, carry):
            par = lax.rem(b, 2)
            nxt = lax.rem(b + 1, 2)

            pltpu.make_async_copy(
                idx2_hbm.at[pl.ds((base + b) * KCH, KCH)], idx2_v.at[par],
                semi).wait()

            @pl.when(b + 1 < SPT)
            def _():
                pltpu.async_copy(
                    idx2_hbm.at[pl.ds((base + b + 1) * KCH, KCH)],
                    idx2_v.at[nxt], semi)

            bcps = [
                pltpu.async_copy(b2_hbm.at[idx2_v.at[par, k]], bsel_v.at[k],
                                 semb)
                for k in range(KCH)
            ]
            pltpu.async_copy(w2_hbm.at[idx2_v.at[par, 0]], rows2_v.at[0],
                             semr[0])

            pltpu.make_async_copy(w1t_hbm.at[idx1_v.at[b]], rows1_v,
                                  sem1).wait()

            v1 = []
            for half in range(2):
                hc = NCH // 2

                def a_body(a0, accs, _half=half, _hc=hc):
                    wvec = inv_v[b, pl.ds(a0 * LANES, LANES)]
                    for j in range(LANES):
                        w = wvec[j]
                        accs = tuple(
                            accs[ci] + w * rows1_v[
                                a0 * LANES + j,
                                pl.ds((_half * _hc + ci) * LANES, LANES)]
                            for ci in range(_hc)
                        )
                    return accs

                zeros = tuple(jnp.zeros((LANES,), jnp.float32)
                              for _ in range(hc))
                accs = plsc.parallel_loop(
                    0, A_IN // LANES, 1, unroll=2, carry=zeros)(a_body)
                for ci in range(hc):
                    cc = half * hc + ci
                    v1.append(jnp.maximum(
                        accs[ci] + b1_v[pl.ds(cc * LANES, LANES)], 0.0))

            for cp in bcps:
                cp.wait()

            @pl.when(b >= 2)
            def _():
                pltpu.make_async_copy(val2_v.at[par],
                                      out_hbm.at[base + b - 2], semo).wait()

            for k in range(KCH):
                if k + 1 < KCH:
                    pltpu.async_copy(w2_hbm.at[idx2_v.at[par, k + 1]],
                                     rows2_v.at[(k + 1) % 2], semr[(k + 1) % 2])
                else:
                    @pl.when(b + 1 < SPT)
                    def _():
                        pltpu.async_copy(w1t_hbm.at[idx1_v.at[b + 1]], rows1_v,
                                         sem1)
                pltpu.make_async_copy(w2_hbm.at[idx2_v.at[par, k]],
                                      rows2_v.at[k % 2], semr[k % 2]).wait()

                def grp_body(g, carry3, _k=k):
                    roff = g * (LANES * LANES)
                    for p in range(LANES):
                        row = g * LANES + p

                        def term(ci):
                            return v1[ci] * rows2_v[
                                _k % 2, row, pl.ds(ci * LANES, LANES)]

                        parts = []
                        for q in range(4):
                            t = term(q)
                            for ci in range(q + 4, NCH, 4):
                                t = t + term(ci)
                            parts.append(t)
                        red_v[pl.ds(roff + p * LANES, LANES)] = _tree_sum(parts)
                    iota16 = iota * LANES + roff
                    cols = [plsc.load_gather(red_v, [iota16 + h])
                            for h in range(LANES)]
                    vs = _tree_sum(cols) + bsel_v[_k, pl.ds(g * LANES, LANES)]
                    val2_v[par, pl.ds(_k * 128 + g * LANES, LANES)] = vs
                    return carry3

                plsc.parallel_loop(0, NG, 1, unroll=2, carry=jnp.int32(0))(
                    grp_body)

            pltpu.async_copy(val2_v.at[par], out_hbm.at[base + b], semo)
            return carry

        lax.fori_loop(0, SPT, sample_body, 0)

        for t in (SPT - 2, SPT - 1):
            pltpu.make_async_copy(val2_v.at[t % 2], out_hbm.at[base + t],
                                  semo).wait()

    return sc_net


def kernel(in_values, active_in_indices, active_label_indices, W1, b1, W2, b2):
    B, A_IN = in_values.shape
    L_OUT = active_label_indices.shape[1]
    H, F = W1.shape
    C = W2.shape[0]

    idx1 = active_in_indices.astype(jnp.int32)
    idx2 = active_label_indices.astype(jnp.int32).reshape(B * (L_OUT // 128), 128)
    w1t = W1.T

    sc_net = _build_sc_net(B, A_IN, L_OUT, H, F, C)
    val2 = sc_net(in_values, idx1, idx2, w1t, b1, W2, b2)
    return (val2, active_label_indices)
